# baseline (speedup 1.0000x reference)
import functools

import jax
import jax.numpy as jnp
from jax import lax
from jax.experimental import pallas as pl
from jax.experimental.pallas import tpu as pltpu
from jax.experimental.pallas import tpu_sc as plsc

N = 10000
D = 128
E_MP = 320000
E_SUP = 100000

NC = 2
NS = 16
NW = NC * NS

CH = 100
NCH = 100
EPW = NCH * CH

N_ACC = 10240
APS = N_ACC // NS
N_OUT = 10240
RPS = N_OUT // NS
ZR = 128
KA = 5

SCH2 = 56
NSCH = 56
SPW = NSCH * SCH2
E_SUP_PAD = NW * SPW


def _mesh():
    return plsc.VectorSubcoreMesh(core_axis_name="c", subcore_axis_name="s")


_SC_PARAMS = pltpu.CompilerParams(use_tc_tiling_on_sc=False)

BF = jnp.bfloat16


def _fill_2d_bf16(buf, rows, cols, val):
    v = jnp.full((32,), val, BF)

    def body(i, carry):
        r = i // (cols // 32)
        col = (i % (cols // 32)) * 32
        buf[r, pl.ds(col, 32)] = v
        return carry

    lax.fori_loop(0, rows * (cols // 32), body, 0)


def _fill_1d(buf, n, val):
    v = jnp.full((16,), val, jnp.float32)

    def body(i, carry):
        buf[pl.ds(i * 16, 16)] = v
        return carry

    lax.fori_loop(0, n // 16, body, 0)


def _zero_acc(zsrc, acc_sh, s):
    base = s * APS
    for k in range(APS // CH):
        pltpu.sync_copy(zsrc, acc_sh.at[pl.ds(base + k * CH, CH)])
    rem = APS % CH
    if rem:
        pltpu.sync_copy(zsrc.at[pl.ds(0, rem)],
                        acc_sh.at[pl.ds(base + (APS // CH) * CH, rem)])


def _agg_pipeline(x_hbm, idx_s, idx_d, bufs_a, bufs_b, acc_sh,
                  gsem, ssem_a, ssem_b, k):
    npairs = NCH // (2 * k)

    def run_group(t, g_off, bufs, ssem):
        g = 2 * t + g_off

        @pl.when(t > 0)
        def _():
            for b in range(k):
                pltpu.make_async_copy(x_hbm.at[pl.ds(0, CH)], bufs[b],
                                      ssem).wait()
        descs = []
        for b in range(k):
            j = g * k + b
            descs.append(
                pltpu.async_copy(x_hbm.at[idx_s.at[j]], bufs[b], gsem))
        for d in descs:
            d.wait()
        for b in range(k):
            j = g * k + b
            pltpu.async_copy(bufs[b], acc_sh.at[idx_d.at[j]], ssem, add=True)

    def pair(t, carry):
        run_group(t, 0, bufs_a, ssem_a)
        run_group(t, 1, bufs_b, ssem_b)
        return carry

    lax.fori_loop(0, npairs, pair, 0)
    for b in range(k):
        pltpu.make_async_copy(x_hbm.at[pl.ds(0, CH)], bufs_a[b], ssem_a).wait()
        pltpu.make_async_copy(x_hbm.at[pl.ds(0, CH)], bufs_b[b], ssem_b).wait()


def _agg_count_body(x_hbm, src_hbm, dst_hbm, part_hbm, cnt_hbm,
                    idx_s, idx_d, zed, ones_v,
                    a0, a1, a2, a3, a4, b0, b1, b2, b3, b4,
                    acc_sh, cnt_sh, gsem, ssem_a, ssem_b, csem):
    c = lax.axis_index("c")
    s = lax.axis_index("s")
    w = c * NS + s

    _fill_1d(zed, APS, 0.0)
    _fill_1d(ones_v, 128, 1.0)
    pltpu.sync_copy(zed, cnt_sh.at[pl.ds(s * APS, APS)])
    _fill_2d_bf16(a0, CH, D, 0.0)
    _zero_acc(a0, acc_sh, s)

    pltpu.sync_copy(src_hbm.at[w], idx_s)
    pltpu.sync_copy(dst_hbm.at[w], idx_d)

    plsc.subcore_barrier()

    def cfire(j, carry):
        pltpu.async_copy(ones_v.at[pl.ds(0, CH)], cnt_sh.at[idx_d.at[j]],
                         csem, add=True)
        return carry

    lax.fori_loop(0, NCH, cfire, 0)

    _agg_pipeline(x_hbm, idx_s, idx_d, (a0, a1, a2, a3, a4),
                  (b0, b1, b2, b3, b4), acc_sh, gsem, ssem_a, ssem_b, KA)

    def cdrain(j, carry):
        pltpu.make_async_copy(cnt_hbm.at[pl.ds(0, CH)], zed.at[pl.ds(0, CH)],
                              csem).wait()
        return carry

    lax.fori_loop(0, NCH, cdrain, 0)

    plsc.subcore_barrier()

    pltpu.sync_copy(acc_sh.at[pl.ds(s * RPS, RPS)],
                    part_hbm.at[c, pl.ds(s * RPS, RPS)])
    pltpu.sync_copy(cnt_sh.at[pl.ds(s * RPS, RPS)],
                    cnt_hbm.at[pl.ds(c * N_OUT + s * RPS, RPS)])


def _agg_body(x_hbm, src_hbm, dst_hbm, part_hbm,
              idx_s, idx_d, a0, a1, a2, a3, a4, b0, b1, b2, b3, b4, acc_sh,
              gsem, ssem_a, ssem_b):
    c = lax.axis_index("c")
    s = lax.axis_index("s")
    w = c * NS + s

    _fill_2d_bf16(a0, CH, D, 0.0)
    _zero_acc(a0, acc_sh, s)

    pltpu.sync_copy(src_hbm.at[w], idx_s)
    pltpu.sync_copy(dst_hbm.at[w], idx_d)

    plsc.subcore_barrier()

    _agg_pipeline(x_hbm, idx_s, idx_d, (a0, a1, a2, a3, a4),
                  (b0, b1, b2, b3, b4), acc_sh, gsem, ssem_a, ssem_b, KA)

    plsc.subcore_barrier()

    pltpu.sync_copy(acc_sh.at[pl.ds(s * RPS, RPS)],
                    part_hbm.at[c, pl.ds(s * RPS, RPS)])


def _row_buf():
    return pltpu.VMEM((CH, D), BF)


_agg_count = functools.partial(
    pl.kernel,
    out_type=[jax.ShapeDtypeStruct((NC, N_OUT, D), BF),
              jax.ShapeDtypeStruct((NC * N_OUT,), jnp.float32)],
    mesh=_mesh(),
    compiler_params=_SC_PARAMS,
    scratch_types=[
        pltpu.VMEM((NCH, CH), jnp.int32),
        pltpu.VMEM((NCH, CH), jnp.int32),
        pltpu.VMEM((APS,), jnp.float32),
        pltpu.VMEM((128,), jnp.float32),
        _row_buf(), _row_buf(), _row_buf(), _row_buf(), _row_buf(),
        _row_buf(), _row_buf(), _row_buf(), _row_buf(), _row_buf(),
        pltpu.VMEM_SHARED((N_ACC, D), BF),
        pltpu.VMEM_SHARED((N_ACC,), jnp.float32),
        pltpu.SemaphoreType.DMA,
        pltpu.SemaphoreType.DMA,
        pltpu.SemaphoreType.DMA,
        pltpu.SemaphoreType.DMA,
    ],
)(_agg_count_body)

_agg = functools.partial(
    pl.kernel,
    out_type=jax.ShapeDtypeStruct((NC, N_OUT, D), BF),
    mesh=_mesh(),
    compiler_params=_SC_PARAMS,
    scratch_types=[
        pltpu.VMEM((NCH, CH), jnp.int32),
        pltpu.VMEM((NCH, CH), jnp.int32),
        _row_buf(), _row_buf(), _row_buf(), _row_buf(), _row_buf(),
        _row_buf(), _row_buf(), _row_buf(), _row_buf(), _row_buf(),
        pltpu.VMEM_SHARED((N_ACC, D), BF),
        pltpu.SemaphoreType.DMA,
        pltpu.SemaphoreType.DMA,
        pltpu.SemaphoreType.DMA,
    ],
)(_agg_body)


def _sup_gather_body(h_hbm, ssrc_hbm, sdst_hbm, osrc_hbm, odst_hbm,
                     idx_s, idx_d, p0s, p0d, p1s, p1d, gsem, wsem_a, wsem_b):
    c = lax.axis_index("c")
    s = lax.axis_index("s")
    w = c * NS + s
    base = w * SPW

    pltpu.sync_copy(ssrc_hbm.at[w], idx_s)
    pltpu.sync_copy(sdst_hbm.at[w], idx_d)

    def run_chunk(t, g_off, bs, bd, wsem):
        j = 2 * t + g_off

        @pl.when(t > 0)
        def _():
            pltpu.make_async_copy(h_hbm.at[pl.ds(0, SCH2)], bs, wsem).wait()
            pltpu.make_async_copy(h_hbm.at[pl.ds(0, SCH2)], bd, wsem).wait()
        d1 = pltpu.async_copy(h_hbm.at[idx_s.at[j]], bs, gsem)
        d2 = pltpu.async_copy(h_hbm.at[idx_d.at[j]], bd, gsem)
        d1.wait()
        d2.wait()
        pltpu.async_copy(bs, osrc_hbm.at[pl.ds(base + j * SCH2, SCH2)], wsem)
        pltpu.async_copy(bd, odst_hbm.at[pl.ds(base + j * SCH2, SCH2)], wsem)

    def pair(t, carry):
        run_chunk(t, 0, p0s, p0d, wsem_a)
        run_chunk(t, 1, p1s, p1d, wsem_b)
        return carry

    lax.fori_loop(0, NSCH // 2, pair, 0)
    pltpu.make_async_copy(h_hbm.at[pl.ds(0, SCH2)], p0s, wsem_a).wait()
    pltpu.make_async_copy(h_hbm.at[pl.ds(0, SCH2)], p0d, wsem_a).wait()
    pltpu.make_async_copy(h_hbm.at[pl.ds(0, SCH2)], p1s, wsem_b).wait()
    pltpu.make_async_copy(h_hbm.at[pl.ds(0, SCH2)], p1d, wsem_b).wait()


_sup_gather = functools.partial(
    pl.kernel,
    out_type=[jax.ShapeDtypeStruct((E_SUP_PAD, D), BF),
              jax.ShapeDtypeStruct((E_SUP_PAD, D), BF)],
    mesh=_mesh(),
    compiler_params=_SC_PARAMS,
    scratch_types=[
        pltpu.VMEM((NSCH, SCH2), jnp.int32),
        pltpu.VMEM((NSCH, SCH2), jnp.int32),
        pltpu.VMEM((SCH2, D), BF),
        pltpu.VMEM((SCH2, D), BF),
        pltpu.VMEM((SCH2, D), BF),
        pltpu.VMEM((SCH2, D), BF),
        pltpu.SemaphoreType.DMA,
        pltpu.SemaphoreType.DMA,
        pltpu.SemaphoreType.DMA,
    ],
)(_sup_gather_body)


_BR = 1000


def _sage_tc(part, cnt2, x, Wl, bl, Wr, relu, emit_bf16):
    def body(p0_ref, p1_ref, cnt_ref, x_ref, wl_ref, bl_ref, wr_ref, *outs):
        cnt = cnt_ref[:, 0] + cnt_ref[:, 1]
        recip = 1.0 / jnp.maximum(cnt, 1.0)
        agg = p0_ref[0].astype(jnp.float32) + p1_ref[0].astype(jnp.float32)
        mean = agg * recip[:, None]
        h = lax.dot_general(mean, wl_ref[...], (((1,), (1,)), ((), ())),
                            preferred_element_type=jnp.float32)
        h = h + bl_ref[...]
        h = h + lax.dot_general(x_ref[...], wr_ref[...], (((1,), (1,)), ((), ())),
                                preferred_element_type=jnp.float32)
        if relu:
            h = jnp.maximum(h, 0.0)
        outs[0][...] = h
        if emit_bf16:
            outs[1][...] = h.astype(BF)

    grid = (N // _BR,)
    out_specs = [pl.BlockSpec((_BR, D), lambda i: (i, 0))]
    out_shape = [jax.ShapeDtypeStruct((N, D), jnp.float32)]
    if emit_bf16:
        out_specs.append(pl.BlockSpec((_BR, D), lambda i: (i, 0)))
        out_shape.append(jax.ShapeDtypeStruct((N, D), BF))
    return pl.pallas_call(
        body,
        grid=grid,
        in_specs=[
            pl.BlockSpec((1, _BR, D), lambda i: (0, i, 0)),
            pl.BlockSpec((1, _BR, D), lambda i: (1, i, 0)),
            pl.BlockSpec((_BR, NC), lambda i: (i, 0)),
            pl.BlockSpec((_BR, D), lambda i: (i, 0)),
            pl.BlockSpec((D, D), lambda i: (0, 0)),
            pl.BlockSpec((1, D), lambda i: (0, 0)),
            pl.BlockSpec((D, D), lambda i: (0, 0)),
        ],
        out_specs=out_specs,
        out_shape=out_shape,
    )(part, part, cnt2, x, Wl, bl, Wr)


_BS = 4000


def _dot_tc(a, b):
    def body(a_ref, b_ref, o_ref):
        prod = a_ref[...].astype(jnp.float32) * b_ref[...].astype(jnp.float32)
        o_ref[...] = jnp.sum(prod, axis=1, keepdims=True)

    grid = (E_SUP // _BS,)
    return pl.pallas_call(
        body,
        grid=grid,
        in_specs=[
            pl.BlockSpec((_BS, D), lambda i: (i, 0)),
            pl.BlockSpec((_BS, D), lambda i: (i, 0)),
        ],
        out_specs=pl.BlockSpec((_BS, 1), lambda i: (i, 0)),
        out_shape=jax.ShapeDtypeStruct((E_SUP, 1), jnp.float32),
    )(a, b)


def kernel(node_embeddings, message_passing_edge_index, supervision_edge_index,
           Wl1, bl1, Wr1, Wl2, bl2, Wr2, Wl3, bl3, Wr3):
    src = message_passing_edge_index[0].reshape(NW, NCH, CH)
    dst = message_passing_edge_index[1].reshape(NW, NCH, CH)
    pad_idx = jnp.arange(E_SUP_PAD - E_SUP, dtype=jnp.int32) % N
    sup_pad = jnp.concatenate(
        [supervision_edge_index, jnp.stack([pad_idx, pad_idx])], axis=1)
    ssrc = sup_pad[0].reshape(NW, NSCH, SCH2)
    sdst = sup_pad[1].reshape(NW, NSCH, SCH2)

    x = node_embeddings
    xb = x.astype(BF)

    part, cnt = _agg_count(xb, src, dst)
    cnt2 = cnt.reshape(NC, N_OUT)[:, :N].T

    h, hb = _sage_tc(part, cnt2, x, Wl1, bl1.reshape(1, D), Wr1, True, True)
    part = _agg(hb, src, dst)
    h, hb = _sage_tc(part, cnt2, h, Wl2, bl2.reshape(1, D), Wr2, True, True)
    part = _agg(hb, src, dst)
    hb = _sage_tc(part, cnt2, h, Wl3, bl3.reshape(1, D), Wr3, False, True)[1]

    src_rows, dst_rows = _sup_gather(hb, ssrc, sdst)
    scores = _dot_tc(src_rows, dst_rows)
    return scores.reshape(E_SUP)


# speedup vs baseline: 1.5928x; 1.5928x over previous
import functools

import jax
import jax.numpy as jnp
from jax import lax
from jax.experimental import pallas as pl
from jax.experimental.pallas import tpu as pltpu
from jax.experimental.pallas import tpu_sc as plsc

N = 10000
D = 128
E_MP = 320000
E_SUP = 100000

NC = 2
NS = 16
NW = NC * NS

CH = 100
NCH = 100
EPW = NCH * CH

N_ACC = 10240
APS = N_ACC // NS
N_OUT = 10240
RPS = N_OUT // NS
ZR = 128
KA = 5

SCH2 = 112
NSCH = 28
SPW = NSCH * SCH2
E_SUP_PAD = NW * SPW


def _mesh():
    return plsc.VectorSubcoreMesh(core_axis_name="c", subcore_axis_name="s")


_SC_PARAMS = pltpu.CompilerParams(use_tc_tiling_on_sc=False)
_SC_PARAMS_NL = pltpu.CompilerParams(use_tc_tiling_on_sc=False,
                                     needs_layout_passes=False)

BF = jnp.bfloat16


def _fill_2d_bf16(buf, rows, cols, val):
    v = jnp.full((32,), val, BF)

    def body(i, carry):
        r = i // (cols // 32)
        col = (i % (cols // 32)) * 32
        buf[r, pl.ds(col, 32)] = v
        return carry

    lax.fori_loop(0, rows * (cols // 32), body, 0)


def _fill_1d(buf, n, val):
    v = jnp.full((16,), val, jnp.float32)

    def body(i, carry):
        buf[pl.ds(i * 16, 16)] = v
        return carry

    lax.fori_loop(0, n // 16, body, 0)


def _zero_acc(zsrc, acc_sh, s):
    base = s * APS
    for k in range(APS // CH):
        pltpu.sync_copy(zsrc, acc_sh.at[pl.ds(base + k * CH, CH)])
    rem = APS % CH
    if rem:
        pltpu.sync_copy(zsrc.at[pl.ds(0, rem)],
                        acc_sh.at[pl.ds(base + (APS // CH) * CH, rem)])


def _agg_pipeline(x_hbm, idx_s, idx_d, bufs_a, bufs_b, acc_sh,
                  gsem, ssem_a, ssem_b, k):
    npairs = NCH // (2 * k)

    def run_group(t, g_off, bufs, ssem):
        g = 2 * t + g_off

        @pl.when(t > 0)
        def _():
            for b in range(k):
                pltpu.make_async_copy(x_hbm.at[pl.ds(0, CH)], bufs[b],
                                      ssem).wait()
        descs = []
        for b in range(k):
            j = g * k + b
            descs.append(
                pltpu.async_copy(x_hbm.at[idx_s.at[j]], bufs[b], gsem))
        for d in descs:
            d.wait()
        for b in range(k):
            j = g * k + b
            pltpu.async_copy(bufs[b], acc_sh.at[idx_d.at[j]], ssem, add=True)

    def pair(t, carry):
        run_group(t, 0, bufs_a, ssem_a)
        run_group(t, 1, bufs_b, ssem_b)
        return carry

    lax.fori_loop(0, npairs, pair, 0)
    for b in range(k):
        pltpu.make_async_copy(x_hbm.at[pl.ds(0, CH)], bufs_a[b], ssem_a).wait()
        pltpu.make_async_copy(x_hbm.at[pl.ds(0, CH)], bufs_b[b], ssem_b).wait()


def _agg_count_body(x_hbm, src_hbm, dst_hbm, part_hbm, cnt_hbm,
                    idx_s, idx_d, zed, ones_v,
                    a0, a1, a2, a3, a4, b0, b1, b2, b3, b4,
                    acc_sh, cnt_sh, gsem, ssem_a, ssem_b, csem):
    c = lax.axis_index("c")
    s = lax.axis_index("s")
    w = c * NS + s

    _fill_1d(zed, APS, 0.0)
    _fill_1d(ones_v, 128, 1.0)
    pltpu.sync_copy(zed, cnt_sh.at[pl.ds(s * APS, APS)])
    _fill_2d_bf16(a0, CH, D, 0.0)
    _zero_acc(a0, acc_sh, s)

    pltpu.sync_copy(src_hbm.at[w], idx_s)
    pltpu.sync_copy(dst_hbm.at[w], idx_d)

    plsc.subcore_barrier()

    def cfire(j, carry):
        pltpu.async_copy(ones_v.at[pl.ds(0, CH)], cnt_sh.at[idx_d.at[j]],
                         csem, add=True)
        return carry

    lax.fori_loop(0, NCH, cfire, 0)

    _agg_pipeline(x_hbm, idx_s, idx_d, (a0, a1, a2, a3, a4),
                  (b0, b1, b2, b3, b4), acc_sh, gsem, ssem_a, ssem_b, KA)

    def cdrain(j, carry):
        pltpu.make_async_copy(cnt_hbm.at[pl.ds(0, CH)], zed.at[pl.ds(0, CH)],
                              csem).wait()
        return carry

    lax.fori_loop(0, NCH, cdrain, 0)

    plsc.subcore_barrier()

    pltpu.sync_copy(acc_sh.at[pl.ds(s * RPS, RPS)],
                    part_hbm.at[c, pl.ds(s * RPS, RPS)])
    pltpu.sync_copy(cnt_sh.at[pl.ds(s * RPS, RPS)],
                    cnt_hbm.at[pl.ds(c * N_OUT + s * RPS, RPS)])


def _agg_body(x_hbm, src_hbm, dst_hbm, part_hbm,
              idx_s, idx_d, a0, a1, a2, a3, a4, b0, b1, b2, b3, b4, acc_sh,
              gsem, ssem_a, ssem_b):
    c = lax.axis_index("c")
    s = lax.axis_index("s")
    w = c * NS + s

    _fill_2d_bf16(a0, CH, D, 0.0)
    _zero_acc(a0, acc_sh, s)

    pltpu.sync_copy(src_hbm.at[w], idx_s)
    pltpu.sync_copy(dst_hbm.at[w], idx_d)

    plsc.subcore_barrier()

    _agg_pipeline(x_hbm, idx_s, idx_d, (a0, a1, a2, a3, a4),
                  (b0, b1, b2, b3, b4), acc_sh, gsem, ssem_a, ssem_b, KA)

    plsc.subcore_barrier()

    pltpu.sync_copy(acc_sh.at[pl.ds(s * RPS, RPS)],
                    part_hbm.at[c, pl.ds(s * RPS, RPS)])


def _row_buf():
    return pltpu.VMEM((CH, D), BF)


_agg_count = functools.partial(
    pl.kernel,
    out_type=[jax.ShapeDtypeStruct((NC, N_OUT, D), BF),
              jax.ShapeDtypeStruct((NC * N_OUT,), jnp.float32)],
    mesh=_mesh(),
    compiler_params=_SC_PARAMS,
    scratch_types=[
        pltpu.VMEM((NCH, CH), jnp.int32),
        pltpu.VMEM((NCH, CH), jnp.int32),
        pltpu.VMEM((APS,), jnp.float32),
        pltpu.VMEM((128,), jnp.float32),
        _row_buf(), _row_buf(), _row_buf(), _row_buf(), _row_buf(),
        _row_buf(), _row_buf(), _row_buf(), _row_buf(), _row_buf(),
        pltpu.VMEM_SHARED((N_ACC, D), BF),
        pltpu.VMEM_SHARED((N_ACC,), jnp.float32),
        pltpu.SemaphoreType.DMA,
        pltpu.SemaphoreType.DMA,
        pltpu.SemaphoreType.DMA,
        pltpu.SemaphoreType.DMA,
    ],
)(_agg_count_body)

_agg = functools.partial(
    pl.kernel,
    out_type=jax.ShapeDtypeStruct((NC, N_OUT, D), BF),
    mesh=_mesh(),
    compiler_params=_SC_PARAMS,
    scratch_types=[
        pltpu.VMEM((NCH, CH), jnp.int32),
        pltpu.VMEM((NCH, CH), jnp.int32),
        _row_buf(), _row_buf(), _row_buf(), _row_buf(), _row_buf(),
        _row_buf(), _row_buf(), _row_buf(), _row_buf(), _row_buf(),
        pltpu.VMEM_SHARED((N_ACC, D), BF),
        pltpu.SemaphoreType.DMA,
        pltpu.SemaphoreType.DMA,
        pltpu.SemaphoreType.DMA,
    ],
)(_agg_body)


def _dot_chunk(bufS, bufD, out_v, j, tbuf):
    row_iota = lax.iota(jnp.int32, 16)

    def grp(g, carry):
        for e_ in range(16):
            e = g * 16 + e_
            acc = jnp.zeros((16,), jnp.float32)
            for k in range(D // 32):
                s32 = bufS[e, pl.ds(k * 32, 32)]
                d32 = bufD[e, pl.ds(k * 32, 32)]
                s0, s1 = plsc.unpack(s32, format=plsc.PackFormat.INTERLEAVED)
                d0, d1 = plsc.unpack(d32, format=plsc.PackFormat.INTERLEAVED)
                acc = acc + s0 * d0 + s1 * d1
            tbuf[e_, pl.ds(0, 16)] = acc
        tot = jnp.zeros((16,), jnp.float32)
        for col in range(16):
            colv = jnp.full((16,), col, jnp.int32)
            tot = tot + plsc.load_gather(tbuf, [row_iota, colv])
        out_v[pl.ds(j * SCH2 + g * 16, 16)] = tot
        return carry

    lax.fori_loop(0, SCH2 // 16, grp, 0)


def _sup_score_body(h_hbm, ssrc_hbm, sdst_hbm, out_hbm,
                    idx_s, idx_d, out_v, tbuf, aS, aD, bS, bD,
                    gsem_a, gsem_b):
    c = lax.axis_index("c")
    s = lax.axis_index("s")
    w = c * NS + s

    pltpu.sync_copy(ssrc_hbm.at[w], idx_s)
    pltpu.sync_copy(sdst_hbm.at[w], idx_d)

    def fire(j, bS_, bD_, sem):
        pltpu.async_copy(h_hbm.at[idx_s.at[j]], bS_, sem)
        pltpu.async_copy(h_hbm.at[idx_d.at[j]], bD_, sem)

    def drain(bS_, bD_, sem):
        pltpu.make_async_copy(h_hbm.at[pl.ds(0, SCH2)], bS_, sem).wait()
        pltpu.make_async_copy(h_hbm.at[pl.ds(0, SCH2)], bD_, sem).wait()

    npairs = NSCH // 2
    fire(0, aS, aD, gsem_a)

    def pair(t, carry):
        fire(2 * t + 1, bS, bD, gsem_b)
        drain(aS, aD, gsem_a)
        _dot_chunk(aS, aD, out_v, 2 * t, tbuf)

        @pl.when(t < npairs - 1)
        def _():
            fire(2 * t + 2, aS, aD, gsem_a)
        drain(bS, bD, gsem_b)
        _dot_chunk(bS, bD, out_v, 2 * t + 1, tbuf)
        return carry

    lax.fori_loop(0, npairs, pair, 0)
    pltpu.sync_copy(out_v, out_hbm.at[pl.ds(w * SPW, SPW)])


_sup_score = functools.partial(
    pl.kernel,
    out_type=jax.ShapeDtypeStruct((E_SUP_PAD,), jnp.float32),
    mesh=_mesh(),
    compiler_params=_SC_PARAMS_NL,
    scratch_types=[
        pltpu.VMEM((NSCH, SCH2), jnp.int32),
        pltpu.VMEM((NSCH, SCH2), jnp.int32),
        pltpu.VMEM((SPW,), jnp.float32),
        pltpu.VMEM((16, 16), jnp.float32),
        pltpu.VMEM((SCH2, D), BF),
        pltpu.VMEM((SCH2, D), BF),
        pltpu.VMEM((SCH2, D), BF),
        pltpu.VMEM((SCH2, D), BF),
        pltpu.SemaphoreType.DMA,
        pltpu.SemaphoreType.DMA,
    ],
)(_sup_score_body)


_BR = 1000


def _sage_tc(part, cnt2, x, Wl, bl, Wr, relu, emit_bf16):
    def body(p0_ref, p1_ref, cnt_ref, x_ref, wl_ref, bl_ref, wr_ref, *outs):
        cnt = cnt_ref[:, 0] + cnt_ref[:, 1]
        recip = 1.0 / jnp.maximum(cnt, 1.0)
        agg = p0_ref[0].astype(jnp.float32) + p1_ref[0].astype(jnp.float32)
        mean = agg * recip[:, None]
        h = lax.dot_general(mean, wl_ref[...], (((1,), (1,)), ((), ())),
                            preferred_element_type=jnp.float32)
        h = h + bl_ref[...]
        h = h + lax.dot_general(x_ref[...], wr_ref[...], (((1,), (1,)), ((), ())),
                                preferred_element_type=jnp.float32)
        if relu:
            h = jnp.maximum(h, 0.0)
        outs[0][...] = h
        if emit_bf16:
            outs[1][...] = h.astype(BF)

    grid = (N // _BR,)
    out_specs = [pl.BlockSpec((_BR, D), lambda i: (i, 0))]
    out_shape = [jax.ShapeDtypeStruct((N, D), jnp.float32)]
    if emit_bf16:
        out_specs.append(pl.BlockSpec((_BR, D), lambda i: (i, 0)))
        out_shape.append(jax.ShapeDtypeStruct((N, D), BF))
    return pl.pallas_call(
        body,
        grid=grid,
        in_specs=[
            pl.BlockSpec((1, _BR, D), lambda i: (0, i, 0)),
            pl.BlockSpec((1, _BR, D), lambda i: (1, i, 0)),
            pl.BlockSpec((_BR, NC), lambda i: (i, 0)),
            pl.BlockSpec((_BR, D), lambda i: (i, 0)),
            pl.BlockSpec((D, D), lambda i: (0, 0)),
            pl.BlockSpec((1, D), lambda i: (0, 0)),
            pl.BlockSpec((D, D), lambda i: (0, 0)),
        ],
        out_specs=out_specs,
        out_shape=out_shape,
    )(part, part, cnt2, x, Wl, bl, Wr)


_BS = 4000


def _dot_tc(a, b):
    def body(a_ref, b_ref, o_ref):
        prod = a_ref[...].astype(jnp.float32) * b_ref[...].astype(jnp.float32)
        o_ref[...] = jnp.sum(prod, axis=1, keepdims=True)

    grid = (E_SUP // _BS,)
    return pl.pallas_call(
        body,
        grid=grid,
        in_specs=[
            pl.BlockSpec((_BS, D), lambda i: (i, 0)),
            pl.BlockSpec((_BS, D), lambda i: (i, 0)),
        ],
        out_specs=pl.BlockSpec((_BS, 1), lambda i: (i, 0)),
        out_shape=jax.ShapeDtypeStruct((E_SUP, 1), jnp.float32),
    )(a, b)


def kernel(node_embeddings, message_passing_edge_index, supervision_edge_index,
           Wl1, bl1, Wr1, Wl2, bl2, Wr2, Wl3, bl3, Wr3):
    src = message_passing_edge_index[0].reshape(NW, NCH, CH)
    dst = message_passing_edge_index[1].reshape(NW, NCH, CH)
    pad_idx = jnp.arange(E_SUP_PAD - E_SUP, dtype=jnp.int32) % N
    sup_pad = jnp.concatenate(
        [supervision_edge_index, jnp.stack([pad_idx, pad_idx])], axis=1)
    ssrc = sup_pad[0].reshape(NW, NSCH, SCH2)
    sdst = sup_pad[1].reshape(NW, NSCH, SCH2)

    x = node_embeddings
    xb = x.astype(BF)

    part, cnt = _agg_count(xb, src, dst)
    cnt2 = cnt.reshape(NC, N_OUT)[:, :N].T

    h, hb = _sage_tc(part, cnt2, x, Wl1, bl1.reshape(1, D), Wr1, True, True)
    part = _agg(hb, src, dst)
    h, hb = _sage_tc(part, cnt2, h, Wl2, bl2.reshape(1, D), Wr2, True, True)
    part = _agg(hb, src, dst)
    hb = _sage_tc(part, cnt2, h, Wl3, bl3.reshape(1, D), Wr3, False, True)[1]

    scores = _sup_score(hb, ssrc, sdst)
    return scores[:E_SUP]


# speedup vs baseline: 1.5930x; 1.0001x over previous
import functools

import jax
import jax.numpy as jnp
from jax import lax
from jax.experimental import pallas as pl
from jax.experimental.pallas import tpu as pltpu
from jax.experimental.pallas import tpu_sc as plsc

N = 10000
D = 128
E_MP = 320000
E_SUP = 100000

NC = 2
NS = 16
NW = NC * NS

CH = 100
NCH = 100
EPW = NCH * CH

N_ACC = 10240
APS = N_ACC // NS
N_OUT = 10240
RPS = N_OUT // NS
ZR = 128
KA = 5

SCH2 = 112
NSCH = 28
SPW = NSCH * SCH2
E_SUP_PAD = NW * SPW


def _mesh():
    return plsc.VectorSubcoreMesh(core_axis_name="c", subcore_axis_name="s")


_SC_PARAMS = pltpu.CompilerParams(use_tc_tiling_on_sc=False)
_SC_PARAMS_NL = pltpu.CompilerParams(use_tc_tiling_on_sc=False,
                                     needs_layout_passes=False)

BF = jnp.bfloat16


def _fill_2d_bf16(buf, rows, cols, val):
    v = jnp.full((32,), val, BF)

    def body(i, carry):
        r = i // (cols // 32)
        col = (i % (cols // 32)) * 32
        buf[r, pl.ds(col, 32)] = v
        return carry

    lax.fori_loop(0, rows * (cols // 32), body, 0)


def _fill_1d(buf, n, val):
    v = jnp.full((16,), val, jnp.float32)

    def body(i, carry):
        buf[pl.ds(i * 16, 16)] = v
        return carry

    lax.fori_loop(0, n // 16, body, 0)


def _zero_acc(zsrc, acc_sh, s):
    base = s * APS
    for k in range(APS // CH):
        pltpu.sync_copy(zsrc, acc_sh.at[pl.ds(base + k * CH, CH)])
    rem = APS % CH
    if rem:
        pltpu.sync_copy(zsrc.at[pl.ds(0, rem)],
                        acc_sh.at[pl.ds(base + (APS // CH) * CH, rem)])


def _agg_pipeline(x_hbm, idx_s, idx_d, bufs_a, bufs_b, acc_sh,
                  gsem, ssem_a, ssem_b, k):
    npairs = NCH // (2 * k)

    def run_group(t, g_off, bufs, ssem):
        g = 2 * t + g_off

        @pl.when(t > 0)
        def _():
            for b in range(k):
                pltpu.make_async_copy(x_hbm.at[pl.ds(0, CH)], bufs[b],
                                      ssem).wait()
        descs = []
        for b in range(k):
            j = g * k + b
            descs.append(
                pltpu.async_copy(x_hbm.at[idx_s.at[j]], bufs[b], gsem))
        for d in descs:
            d.wait()
        for b in range(k):
            j = g * k + b
            pltpu.async_copy(bufs[b], acc_sh.at[idx_d.at[j]], ssem, add=True)

    def pair(t, carry):
        run_group(t, 0, bufs_a, ssem_a)
        run_group(t, 1, bufs_b, ssem_b)
        return carry

    lax.fori_loop(0, npairs, pair, 0)
    for b in range(k):
        pltpu.make_async_copy(x_hbm.at[pl.ds(0, CH)], bufs_a[b], ssem_a).wait()
        pltpu.make_async_copy(x_hbm.at[pl.ds(0, CH)], bufs_b[b], ssem_b).wait()


def _agg_count_body(x_hbm, src_hbm, dst_hbm, part_hbm, cnt_hbm,
                    idx_s, idx_d, zed, ones_v,
                    a0, a1, a2, a3, a4, b0, b1, b2, b3, b4,
                    acc_sh, cnt_sh, gsem, ssem_a, ssem_b, csem):
    c = lax.axis_index("c")
    s = lax.axis_index("s")
    w = c * NS + s

    _fill_1d(zed, APS, 0.0)
    _fill_1d(ones_v, 128, 1.0)
    pltpu.sync_copy(zed, cnt_sh.at[pl.ds(s * APS, APS)])
    _fill_2d_bf16(a0, CH, D, 0.0)
    _zero_acc(a0, acc_sh, s)

    pltpu.sync_copy(src_hbm.at[w], idx_s)
    pltpu.sync_copy(dst_hbm.at[w], idx_d)

    plsc.subcore_barrier()

    def cfire(j, carry):
        pltpu.async_copy(ones_v.at[pl.ds(0, CH)], cnt_sh.at[idx_d.at[j]],
                         csem, add=True)
        return carry

    lax.fori_loop(0, NCH, cfire, 0)

    _agg_pipeline(x_hbm, idx_s, idx_d, (a0, a1, a2, a3, a4),
                  (b0, b1, b2, b3, b4), acc_sh, gsem, ssem_a, ssem_b, KA)

    def cdrain(j, carry):
        pltpu.make_async_copy(cnt_hbm.at[pl.ds(0, CH)], zed.at[pl.ds(0, CH)],
                              csem).wait()
        return carry

    lax.fori_loop(0, NCH, cdrain, 0)

    plsc.subcore_barrier()

    pltpu.sync_copy(acc_sh.at[pl.ds(s * RPS, RPS)],
                    part_hbm.at[c, pl.ds(s * RPS, RPS)])
    pltpu.sync_copy(cnt_sh.at[pl.ds(s * RPS, RPS)],
                    cnt_hbm.at[pl.ds(c * N_OUT + s * RPS, RPS)])


def _agg_body(x_hbm, src_hbm, dst_hbm, part_hbm,
              idx_s, idx_d, a0, a1, a2, a3, a4, b0, b1, b2, b3, b4, acc_sh,
              gsem, ssem_a, ssem_b):
    c = lax.axis_index("c")
    s = lax.axis_index("s")
    w = c * NS + s

    _fill_2d_bf16(a0, CH, D, 0.0)
    _zero_acc(a0, acc_sh, s)

    pltpu.sync_copy(src_hbm.at[w], idx_s)
    pltpu.sync_copy(dst_hbm.at[w], idx_d)

    plsc.subcore_barrier()

    _agg_pipeline(x_hbm, idx_s, idx_d, (a0, a1, a2, a3, a4),
                  (b0, b1, b2, b3, b4), acc_sh, gsem, ssem_a, ssem_b, KA)

    plsc.subcore_barrier()

    pltpu.sync_copy(acc_sh.at[pl.ds(s * RPS, RPS)],
                    part_hbm.at[c, pl.ds(s * RPS, RPS)])


def _row_buf():
    return pltpu.VMEM((CH, D), BF)


_agg_count = functools.partial(
    pl.kernel,
    out_type=[jax.ShapeDtypeStruct((NC, N_OUT, D), BF),
              jax.ShapeDtypeStruct((NC * N_OUT,), jnp.float32)],
    mesh=_mesh(),
    compiler_params=_SC_PARAMS,
    scratch_types=[
        pltpu.VMEM((NCH, CH), jnp.int32),
        pltpu.VMEM((NCH, CH), jnp.int32),
        pltpu.VMEM((APS,), jnp.float32),
        pltpu.VMEM((128,), jnp.float32),
        _row_buf(), _row_buf(), _row_buf(), _row_buf(), _row_buf(),
        _row_buf(), _row_buf(), _row_buf(), _row_buf(), _row_buf(),
        pltpu.VMEM_SHARED((N_ACC, D), BF),
        pltpu.VMEM_SHARED((N_ACC,), jnp.float32),
        pltpu.SemaphoreType.DMA,
        pltpu.SemaphoreType.DMA,
        pltpu.SemaphoreType.DMA,
        pltpu.SemaphoreType.DMA,
    ],
)(_agg_count_body)

_agg = functools.partial(
    pl.kernel,
    out_type=jax.ShapeDtypeStruct((NC, N_OUT, D), BF),
    mesh=_mesh(),
    compiler_params=_SC_PARAMS,
    scratch_types=[
        pltpu.VMEM((NCH, CH), jnp.int32),
        pltpu.VMEM((NCH, CH), jnp.int32),
        _row_buf(), _row_buf(), _row_buf(), _row_buf(), _row_buf(),
        _row_buf(), _row_buf(), _row_buf(), _row_buf(), _row_buf(),
        pltpu.VMEM_SHARED((N_ACC, D), BF),
        pltpu.SemaphoreType.DMA,
        pltpu.SemaphoreType.DMA,
        pltpu.SemaphoreType.DMA,
    ],
)(_agg_body)


def _dot_chunk(bufS, bufD, out_v, j, tbuf):
    row_iota = lax.iota(jnp.int32, 16)

    def grp(g, carry):
        for e_ in range(16):
            e = g * 16 + e_
            acc = jnp.zeros((16,), jnp.float32)
            for k in range(D // 32):
                s32 = bufS[e, pl.ds(k * 32, 32)]
                d32 = bufD[e, pl.ds(k * 32, 32)]
                s0, s1 = plsc.unpack(s32, format=plsc.PackFormat.INTERLEAVED)
                d0, d1 = plsc.unpack(d32, format=plsc.PackFormat.INTERLEAVED)
                acc = acc + s0 * d0 + s1 * d1
            tbuf[e_, pl.ds(0, 16)] = acc
        tot = jnp.zeros((16,), jnp.float32)
        for col in range(16):
            colv = jnp.full((16,), col, jnp.int32)
            tot = tot + plsc.load_gather(tbuf, [row_iota, colv])
        out_v[pl.ds(j * SCH2 + g * 16, 16)] = tot
        return carry

    lax.fori_loop(0, SCH2 // 16, grp, 0)


def _sup_score_body(h_hbm, ssrc_hbm, sdst_hbm, out_hbm,
                    idx_s, idx_d, out_v, tbuf, aS, aD, bS, bD,
                    gsem_a, gsem_b):
    c = lax.axis_index("c")
    s = lax.axis_index("s")
    w = c * NS + s

    pltpu.sync_copy(ssrc_hbm.at[w], idx_s)
    pltpu.sync_copy(sdst_hbm.at[w], idx_d)

    def fire(j, bS_, bD_, sem):
        pltpu.async_copy(h_hbm.at[idx_s.at[j]], bS_, sem)
        pltpu.async_copy(h_hbm.at[idx_d.at[j]], bD_, sem)

    def drain(bS_, bD_, sem):
        pltpu.make_async_copy(h_hbm.at[pl.ds(0, SCH2)], bS_, sem).wait()
        pltpu.make_async_copy(h_hbm.at[pl.ds(0, SCH2)], bD_, sem).wait()

    npairs = NSCH // 2
    fire(0, aS, aD, gsem_a)

    def pair(t, carry):
        fire(2 * t + 1, bS, bD, gsem_b)
        drain(aS, aD, gsem_a)
        _dot_chunk(aS, aD, out_v, 2 * t, tbuf)

        @pl.when(t < npairs - 1)
        def _():
            fire(2 * t + 2, aS, aD, gsem_a)
        drain(bS, bD, gsem_b)
        _dot_chunk(bS, bD, out_v, 2 * t + 1, tbuf)
        return carry

    lax.fori_loop(0, npairs, pair, 0)
    pltpu.sync_copy(out_v, out_hbm.at[pl.ds(w * SPW, SPW)])


_sup_score = functools.partial(
    pl.kernel,
    out_type=jax.ShapeDtypeStruct((E_SUP_PAD,), jnp.float32),
    mesh=_mesh(),
    compiler_params=_SC_PARAMS_NL,
    scratch_types=[
        pltpu.VMEM((NSCH, SCH2), jnp.int32),
        pltpu.VMEM((NSCH, SCH2), jnp.int32),
        pltpu.VMEM((SPW,), jnp.float32),
        pltpu.VMEM((16, 16), jnp.float32),
        pltpu.VMEM((SCH2, D), BF),
        pltpu.VMEM((SCH2, D), BF),
        pltpu.VMEM((SCH2, D), BF),
        pltpu.VMEM((SCH2, D), BF),
        pltpu.SemaphoreType.DMA,
        pltpu.SemaphoreType.DMA,
    ],
)(_sup_score_body)


def _sage_tc(part, cnt2, x, Wl, bl, Wr, relu, emit_bf16):
    def body(part_any, cnt_ref, x_ref, wl_ref, bl_ref, wr_ref, *rest):
        if emit_bf16:
            h_out, hb_any, pvm, hbvm = rest
        else:
            h_out, pvm = rest
        pltpu.sync_copy(part_any, pvm)
        cnt = cnt_ref[:, 0] + cnt_ref[:, 1]
        recip = 1.0 / jnp.maximum(cnt, 1.0)
        agg = (pvm[0, :N, :].astype(jnp.float32)
               + pvm[1, :N, :].astype(jnp.float32))
        mean = agg * recip[:, None]
        h = lax.dot_general(mean, wl_ref[...], (((1,), (1,)), ((), ())),
                            preferred_element_type=jnp.float32)
        h = h + bl_ref[...]
        h = h + lax.dot_general(x_ref[...], wr_ref[...], (((1,), (1,)), ((), ())),
                                preferred_element_type=jnp.float32)
        if relu:
            h = jnp.maximum(h, 0.0)
        h_out[...] = h
        if emit_bf16:
            hbvm[...] = h.astype(BF)
            pltpu.sync_copy(hbvm, hb_any)

    out_specs = [pl.BlockSpec((N, D), lambda: (0, 0))]
    out_shape = [jax.ShapeDtypeStruct((N, D), jnp.float32)]
    scratch = [pltpu.VMEM((NC, N_OUT, D), BF)]
    if emit_bf16:
        out_specs.append(pl.BlockSpec(memory_space=pl.ANY))
        out_shape.append(jax.ShapeDtypeStruct((N, D), BF))
        scratch.append(pltpu.VMEM((N, D), BF))
    return pl.pallas_call(
        body,
        in_specs=[
            pl.BlockSpec(memory_space=pl.ANY),
            pl.BlockSpec((N, NC), lambda: (0, 0)),
            pl.BlockSpec((N, D), lambda: (0, 0)),
            pl.BlockSpec((D, D), lambda: (0, 0)),
            pl.BlockSpec((1, D), lambda: (0, 0)),
            pl.BlockSpec((D, D), lambda: (0, 0)),
        ],
        out_specs=out_specs,
        out_shape=out_shape,
        scratch_shapes=scratch,
    )(part, cnt2, x, Wl, bl, Wr)


_BS = 4000


def _dot_tc(a, b):
    def body(a_ref, b_ref, o_ref):
        prod = a_ref[...].astype(jnp.float32) * b_ref[...].astype(jnp.float32)
        o_ref[...] = jnp.sum(prod, axis=1, keepdims=True)

    grid = (E_SUP // _BS,)
    return pl.pallas_call(
        body,
        grid=grid,
        in_specs=[
            pl.BlockSpec((_BS, D), lambda i: (i, 0)),
            pl.BlockSpec((_BS, D), lambda i: (i, 0)),
        ],
        out_specs=pl.BlockSpec((_BS, 1), lambda i: (i, 0)),
        out_shape=jax.ShapeDtypeStruct((E_SUP, 1), jnp.float32),
    )(a, b)


def kernel(node_embeddings, message_passing_edge_index, supervision_edge_index,
           Wl1, bl1, Wr1, Wl2, bl2, Wr2, Wl3, bl3, Wr3):
    src = message_passing_edge_index[0].reshape(NW, NCH, CH)
    dst = message_passing_edge_index[1].reshape(NW, NCH, CH)
    pad_idx = jnp.arange(E_SUP_PAD - E_SUP, dtype=jnp.int32) % N
    sup_pad = jnp.concatenate(
        [supervision_edge_index, jnp.stack([pad_idx, pad_idx])], axis=1)
    ssrc = sup_pad[0].reshape(NW, NSCH, SCH2)
    sdst = sup_pad[1].reshape(NW, NSCH, SCH2)

    x = node_embeddings
    xb = x.astype(BF)

    part, cnt = _agg_count(xb, src, dst)
    cnt2 = cnt.reshape(NC, N_OUT)[:, :N].T

    h, hb = _sage_tc(part, cnt2, x, Wl1, bl1.reshape(1, D), Wr1, True, True)
    part = _agg(hb, src, dst)
    h, hb = _sage_tc(part, cnt2, h, Wl2, bl2.reshape(1, D), Wr2, True, True)
    part = _agg(hb, src, dst)
    hb = _sage_tc(part, cnt2, h, Wl3, bl3.reshape(1, D), Wr3, False, True)[1]

    scores = _sup_score(hb, ssrc, sdst)
    return scores[:E_SUP]


# speedup vs baseline: 1.5950x; 1.0013x over previous
import functools

import jax
import jax.numpy as jnp
from jax import lax
from jax.experimental import pallas as pl
from jax.experimental.pallas import tpu as pltpu
from jax.experimental.pallas import tpu_sc as plsc

N = 10000
D = 128
E_MP = 320000
E_SUP = 100000

NC = 2
NS = 16
NW = NC * NS

CH = 100
NCH = 100
EPW = NCH * CH

N_ACC = 10240
APS = N_ACC // NS
N_OUT = 10240
RPS = N_OUT // NS
ZR = 128
KA = 5

SCH2 = 112
NSCH = 28
SPW = NSCH * SCH2
E_SUP_PAD = NW * SPW


def _mesh():
    return plsc.VectorSubcoreMesh(core_axis_name="c", subcore_axis_name="s")


_SC_PARAMS = pltpu.CompilerParams(use_tc_tiling_on_sc=False)
_SC_PARAMS_NL = pltpu.CompilerParams(use_tc_tiling_on_sc=False,
                                     needs_layout_passes=False)

BF = jnp.bfloat16


def _fill_2d_bf16(buf, rows, cols, val):
    v = jnp.full((32,), val, BF)

    def body(i, carry):
        r = i // (cols // 32)
        col = (i % (cols // 32)) * 32
        buf[r, pl.ds(col, 32)] = v
        return carry

    lax.fori_loop(0, rows * (cols // 32), body, 0)


def _fill_1d(buf, n, val):
    v = jnp.full((16,), val, jnp.float32)

    def body(i, carry):
        buf[pl.ds(i * 16, 16)] = v
        return carry

    lax.fori_loop(0, n // 16, body, 0)


def _zero_acc(zsrc, acc_sh, s):
    base = s * APS
    for k in range(APS // CH):
        pltpu.sync_copy(zsrc, acc_sh.at[pl.ds(base + k * CH, CH)])
    rem = APS % CH
    if rem:
        pltpu.sync_copy(zsrc.at[pl.ds(0, rem)],
                        acc_sh.at[pl.ds(base + (APS // CH) * CH, rem)])


def _agg_pipeline(x_hbm, idx_s, idx_d, bufs_a, bufs_b, acc_sh,
                  gsem, ssem_a, ssem_b, k):
    npairs = NCH // (2 * k)

    def run_group(t, g_off, bufs, ssem):
        g = 2 * t + g_off

        @pl.when(t > 0)
        def _():
            for b in range(k):
                pltpu.make_async_copy(x_hbm.at[pl.ds(0, CH)], bufs[b],
                                      ssem).wait()
        descs = []
        for b in range(k):
            j = g * k + b
            descs.append(
                pltpu.async_copy(x_hbm.at[idx_s.at[j]], bufs[b], gsem))
        for d in descs:
            d.wait()
        for b in range(k):
            j = g * k + b
            pltpu.async_copy(bufs[b], acc_sh.at[idx_d.at[j]], ssem, add=True)

    def pair(t, carry):
        run_group(t, 0, bufs_a, ssem_a)
        run_group(t, 1, bufs_b, ssem_b)
        return carry

    lax.fori_loop(0, npairs, pair, 0)
    for b in range(k):
        pltpu.make_async_copy(x_hbm.at[pl.ds(0, CH)], bufs_a[b], ssem_a).wait()
        pltpu.make_async_copy(x_hbm.at[pl.ds(0, CH)], bufs_b[b], ssem_b).wait()


def _agg_count_body(x_hbm, src_hbm, dst_hbm, part_hbm, cnt_hbm,
                    idx_s, idx_d, zed, ones_v,
                    a0, a1, a2, a3, a4, b0, b1, b2, b3, b4,
                    acc_sh, cnt_sh, gsem, ssem_a, ssem_b, csem):
    c = lax.axis_index("c")
    s = lax.axis_index("s")
    w = c * NS + s

    _fill_1d(zed, APS, 0.0)
    _fill_1d(ones_v, 128, 1.0)
    pltpu.sync_copy(zed, cnt_sh.at[pl.ds(s * APS, APS)])
    _fill_2d_bf16(a0, CH, D, 0.0)
    _zero_acc(a0, acc_sh, s)

    pltpu.sync_copy(src_hbm.at[w], idx_s)
    pltpu.sync_copy(dst_hbm.at[w], idx_d)

    plsc.subcore_barrier()

    def cfire(j, carry):
        pltpu.async_copy(ones_v.at[pl.ds(0, CH)], cnt_sh.at[idx_d.at[j]],
                         csem, add=True)
        return carry

    lax.fori_loop(0, NCH, cfire, 0)

    _agg_pipeline(x_hbm, idx_s, idx_d, (a0, a1, a2, a3, a4),
                  (b0, b1, b2, b3, b4), acc_sh, gsem, ssem_a, ssem_b, KA)

    def cdrain(j, carry):
        pltpu.make_async_copy(cnt_hbm.at[pl.ds(0, CH)], zed.at[pl.ds(0, CH)],
                              csem).wait()
        return carry

    lax.fori_loop(0, NCH, cdrain, 0)

    plsc.subcore_barrier()

    pltpu.sync_copy(acc_sh.at[pl.ds(s * RPS, RPS)],
                    part_hbm.at[c, pl.ds(s * RPS, RPS)])
    pltpu.sync_copy(cnt_sh.at[pl.ds(s * RPS, RPS)],
                    cnt_hbm.at[pl.ds(c * N_OUT + s * RPS, RPS)])


def _agg_body(x_hbm, src_hbm, dst_hbm, part_hbm,
              idx_s, idx_d, a0, a1, a2, a3, a4, b0, b1, b2, b3, b4, acc_sh,
              gsem, ssem_a, ssem_b):
    c = lax.axis_index("c")
    s = lax.axis_index("s")
    w = c * NS + s

    _fill_2d_bf16(a0, CH, D, 0.0)
    _zero_acc(a0, acc_sh, s)

    pltpu.sync_copy(src_hbm.at[w], idx_s)
    pltpu.sync_copy(dst_hbm.at[w], idx_d)

    plsc.subcore_barrier()

    _agg_pipeline(x_hbm, idx_s, idx_d, (a0, a1, a2, a3, a4),
                  (b0, b1, b2, b3, b4), acc_sh, gsem, ssem_a, ssem_b, KA)

    plsc.subcore_barrier()

    pltpu.sync_copy(acc_sh.at[pl.ds(s * RPS, RPS)],
                    part_hbm.at[c, pl.ds(s * RPS, RPS)])


def _row_buf():
    return pltpu.VMEM((CH, D), BF)


_agg_count = functools.partial(
    pl.kernel,
    out_type=[jax.ShapeDtypeStruct((NC, N_OUT, D), BF),
              jax.ShapeDtypeStruct((NC * N_OUT,), jnp.float32)],
    mesh=_mesh(),
    compiler_params=_SC_PARAMS,
    scratch_types=[
        pltpu.VMEM((NCH, CH), jnp.int32),
        pltpu.VMEM((NCH, CH), jnp.int32),
        pltpu.VMEM((APS,), jnp.float32),
        pltpu.VMEM((128,), jnp.float32),
        _row_buf(), _row_buf(), _row_buf(), _row_buf(), _row_buf(),
        _row_buf(), _row_buf(), _row_buf(), _row_buf(), _row_buf(),
        pltpu.VMEM_SHARED((N_ACC, D), BF),
        pltpu.VMEM_SHARED((N_ACC,), jnp.float32),
        pltpu.SemaphoreType.DMA,
        pltpu.SemaphoreType.DMA,
        pltpu.SemaphoreType.DMA,
        pltpu.SemaphoreType.DMA,
    ],
)(_agg_count_body)

_agg = functools.partial(
    pl.kernel,
    out_type=jax.ShapeDtypeStruct((NC, N_OUT, D), BF),
    mesh=_mesh(),
    compiler_params=_SC_PARAMS,
    scratch_types=[
        pltpu.VMEM((NCH, CH), jnp.int32),
        pltpu.VMEM((NCH, CH), jnp.int32),
        _row_buf(), _row_buf(), _row_buf(), _row_buf(), _row_buf(),
        _row_buf(), _row_buf(), _row_buf(), _row_buf(), _row_buf(),
        pltpu.VMEM_SHARED((N_ACC, D), BF),
        pltpu.SemaphoreType.DMA,
        pltpu.SemaphoreType.DMA,
        pltpu.SemaphoreType.DMA,
    ],
)(_agg_body)


def _dot_chunk(bufS, bufD, out_v, j, tbuf):
    row_iota = lax.iota(jnp.int32, 16)

    def grp(g, carry):
        for e_ in range(16):
            e = g * 16 + e_
            acc = jnp.zeros((16,), jnp.float32)
            for k in range(D // 32):
                s32 = bufS[e, pl.ds(k * 32, 32)]
                d32 = bufD[e, pl.ds(k * 32, 32)]
                s0, s1 = plsc.unpack(s32, format=plsc.PackFormat.INTERLEAVED)
                d0, d1 = plsc.unpack(d32, format=plsc.PackFormat.INTERLEAVED)
                acc = acc + s0 * d0 + s1 * d1
            tbuf[e_, pl.ds(0, 16)] = acc
        tot = jnp.zeros((16,), jnp.float32)
        for col in range(16):
            colv = jnp.full((16,), col, jnp.int32)
            tot = tot + plsc.load_gather(tbuf, [row_iota, colv])
        out_v[pl.ds(j * SCH2 + g * 16, 16)] = tot
        return carry

    lax.fori_loop(0, SCH2 // 16, grp, 0)


def _sup_score_body(h_hbm, ssrc_hbm, sdst_hbm, out_hbm,
                    idx_s, idx_d, out_v, tbuf, aS, aD, bS, bD,
                    gsem_a, gsem_b):
    c = lax.axis_index("c")
    s = lax.axis_index("s")
    w = c * NS + s

    pltpu.sync_copy(ssrc_hbm.at[w], idx_s)
    pltpu.sync_copy(sdst_hbm.at[w], idx_d)

    def fire(j, bS_, bD_, sem):
        pltpu.async_copy(h_hbm.at[idx_s.at[j]], bS_, sem)
        pltpu.async_copy(h_hbm.at[idx_d.at[j]], bD_, sem)

    def drain(bS_, bD_, sem):
        pltpu.make_async_copy(h_hbm.at[pl.ds(0, SCH2)], bS_, sem).wait()
        pltpu.make_async_copy(h_hbm.at[pl.ds(0, SCH2)], bD_, sem).wait()

    npairs = NSCH // 2
    fire(0, aS, aD, gsem_a)

    def pair(t, carry):
        fire(2 * t + 1, bS, bD, gsem_b)
        drain(aS, aD, gsem_a)
        _dot_chunk(aS, aD, out_v, 2 * t, tbuf)

        @pl.when(t < npairs - 1)
        def _():
            fire(2 * t + 2, aS, aD, gsem_a)
        drain(bS, bD, gsem_b)
        _dot_chunk(bS, bD, out_v, 2 * t + 1, tbuf)
        return carry

    lax.fori_loop(0, npairs, pair, 0)
    pltpu.sync_copy(out_v, out_hbm.at[pl.ds(w * SPW, SPW)])


_sup_score = functools.partial(
    pl.kernel,
    out_type=jax.ShapeDtypeStruct((E_SUP_PAD,), jnp.float32),
    mesh=_mesh(),
    compiler_params=_SC_PARAMS_NL,
    scratch_types=[
        pltpu.VMEM((NSCH, SCH2), jnp.int32),
        pltpu.VMEM((NSCH, SCH2), jnp.int32),
        pltpu.VMEM((SPW,), jnp.float32),
        pltpu.VMEM((16, 16), jnp.float32),
        pltpu.VMEM((SCH2, D), BF),
        pltpu.VMEM((SCH2, D), BF),
        pltpu.VMEM((SCH2, D), BF),
        pltpu.VMEM((SCH2, D), BF),
        pltpu.SemaphoreType.DMA,
        pltpu.SemaphoreType.DMA,
    ],
)(_sup_score_body)


def _sage_tc(part, cnt2, x, Wl, bl, Wr, relu, emit_bf16):
    def body(part_any, cnt_ref, x_ref, wl_ref, bl_ref, wr_ref, *rest):
        if emit_bf16:
            h_out, hb_any, pvm, hbvm = rest
        else:
            h_out, pvm = rest
        pltpu.sync_copy(part_any, pvm)
        cnt = cnt_ref[:, 0] + cnt_ref[:, 1]
        recip = 1.0 / jnp.maximum(cnt, 1.0)
        agg = (pvm[0, :N, :].astype(jnp.float32)
               + pvm[1, :N, :].astype(jnp.float32))
        mean = agg * recip[:, None]
        h = lax.dot_general(mean, wl_ref[...], (((1,), (1,)), ((), ())),
                            preferred_element_type=jnp.float32)
        h = h + bl_ref[...]
        h = h + lax.dot_general(x_ref[...], wr_ref[...], (((1,), (1,)), ((), ())),
                                preferred_element_type=jnp.float32)
        if relu:
            h = jnp.maximum(h, 0.0)
        h_out[...] = h
        if emit_bf16:
            hbvm[...] = h.astype(BF)
            pltpu.sync_copy(hbvm, hb_any)

    out_specs = [pl.BlockSpec((N, D), lambda: (0, 0))]
    out_shape = [jax.ShapeDtypeStruct((N, D), jnp.float32)]
    scratch = [pltpu.VMEM((NC, N_OUT, D), BF)]
    if emit_bf16:
        out_specs.append(pl.BlockSpec(memory_space=pl.ANY))
        out_shape.append(jax.ShapeDtypeStruct((N, D), BF))
        scratch.append(pltpu.VMEM((N, D), BF))
    return pl.pallas_call(
        body,
        in_specs=[
            pl.BlockSpec(memory_space=pl.ANY),
            pl.BlockSpec((N, NC), lambda: (0, 0)),
            pl.BlockSpec((N, D), lambda: (0, 0)),
            pl.BlockSpec((D, D), lambda: (0, 0)),
            pl.BlockSpec((1, D), lambda: (0, 0)),
            pl.BlockSpec((D, D), lambda: (0, 0)),
        ],
        out_specs=out_specs,
        out_shape=out_shape,
        scratch_shapes=scratch,
    )(part, cnt2, x, Wl, bl, Wr)


_BS = 4000


def _dot_tc(a, b):
    def body(a_ref, b_ref, o_ref):
        prod = a_ref[...].astype(jnp.float32) * b_ref[...].astype(jnp.float32)
        o_ref[...] = jnp.sum(prod, axis=1, keepdims=True)

    grid = (E_SUP // _BS,)
    return pl.pallas_call(
        body,
        grid=grid,
        in_specs=[
            pl.BlockSpec((_BS, D), lambda i: (i, 0)),
            pl.BlockSpec((_BS, D), lambda i: (i, 0)),
        ],
        out_specs=pl.BlockSpec((_BS, 1), lambda i: (i, 0)),
        out_shape=jax.ShapeDtypeStruct((E_SUP, 1), jnp.float32),
    )(a, b)


def kernel(node_embeddings, message_passing_edge_index, supervision_edge_index,
           Wl1, bl1, Wr1, Wl2, bl2, Wr2, Wl3, bl3, Wr3):
    src = message_passing_edge_index[0].reshape(NW, NCH, CH)
    dst = message_passing_edge_index[1].reshape(NW, NCH, CH)
    pad_idx = jnp.arange(E_SUP_PAD - E_SUP, dtype=jnp.int32) % N
    sup_pad = jnp.concatenate(
        [supervision_edge_index, jnp.stack([pad_idx, pad_idx])], axis=1)
    ssrc = sup_pad[0].reshape(NW, NSCH, SCH2)
    sdst = sup_pad[1].reshape(NW, NSCH, SCH2)

    x = node_embeddings
    xb = x.astype(BF)

    part, cnt = _agg_count(xb, src, dst)
    cnt2 = jax.lax.optimization_barrier(cnt.reshape(NC, N_OUT)[:, :N].T)

    h, hb = _sage_tc(part, cnt2, x, Wl1, bl1.reshape(1, D), Wr1, True, True)
    part = _agg(hb, src, dst)
    h, hb = _sage_tc(part, cnt2, h, Wl2, bl2.reshape(1, D), Wr2, True, True)
    part = _agg(hb, src, dst)
    hb = _sage_tc(part, cnt2, h, Wl3, bl3.reshape(1, D), Wr3, False, True)[1]

    scores = _sup_score(hb, ssrc, sdst)
    return scores[:E_SUP]


# speedup vs baseline: 1.5971x; 1.0013x over previous
import functools

import jax
import jax.numpy as jnp
from jax import lax
from jax.experimental import pallas as pl
from jax.experimental.pallas import tpu as pltpu
from jax.experimental.pallas import tpu_sc as plsc

N = 10000
D = 128
E_MP = 320000
E_SUP = 100000

NC = 2
NS = 16
NW = NC * NS

CH = 100
NCH = 100
EPW = NCH * CH

N_ACC = 10240
APS = N_ACC // NS
N_OUT = 10240
RPS = N_OUT // NS
KA = 5

SCH2 = 112
NSCH = 28
SPW = NSCH * SCH2
E_SUP_PAD = NW * SPW


def _mesh():
    return plsc.VectorSubcoreMesh(core_axis_name="c", subcore_axis_name="s")


_SC_PARAMS = pltpu.CompilerParams(use_tc_tiling_on_sc=False)
_SC_PARAMS_NL = pltpu.CompilerParams(use_tc_tiling_on_sc=False,
                                     needs_layout_passes=False)

BF = jnp.bfloat16


def _fill_2d_bf16(buf, rows, cols, val):
    v = jnp.full((2, 16), val, BF)

    def body(i, carry):
        r = pl.multiple_of(2 * (i // (cols // 16)), 2)
        col = (i % (cols // 16)) * 16
        buf[pl.ds(r, 2), pl.ds(col, 16)] = v
        return carry

    lax.fori_loop(0, (rows // 2) * (cols // 16), body, 0)


def _fill_1d(buf, n, val):
    v = jnp.full((16,), val, jnp.float32)

    def body(i, carry):
        buf[pl.ds(i * 16, 16)] = v
        return carry

    lax.fori_loop(0, n // 16, body, 0)


def _zero_acc(zsrc, acc_sh, s):
    base = s * APS
    for k in range(APS // CH):
        pltpu.sync_copy(zsrc, acc_sh.at[pl.ds(base + k * CH, CH)])
    rem = APS % CH
    if rem:
        pltpu.sync_copy(zsrc.at[pl.ds(0, rem)],
                        acc_sh.at[pl.ds(base + (APS // CH) * CH, rem)])


def _agg_pipeline(x_hbm, idx_s, idx_d, bufs_a, bufs_b, acc_sh,
                  gsem, ssem_a, ssem_b, k):
    npairs = NCH // (2 * k)

    def run_group(t, g_off, bufs, ssem):
        g = 2 * t + g_off

        @pl.when(t > 0)
        def _():
            for b in range(k):
                pltpu.make_async_copy(x_hbm.at[pl.ds(0, CH)], bufs[b],
                                      ssem).wait()
        descs = []
        for b in range(k):
            j = g * k + b
            descs.append(
                pltpu.async_copy(x_hbm.at[idx_s.at[j]], bufs[b], gsem))
        for d in descs:
            d.wait()
        for b in range(k):
            j = g * k + b
            pltpu.async_copy(bufs[b], acc_sh.at[idx_d.at[j]], ssem, add=True)

    def pair(t, carry):
        run_group(t, 0, bufs_a, ssem_a)
        run_group(t, 1, bufs_b, ssem_b)
        return carry

    lax.fori_loop(0, npairs, pair, 0)
    for b in range(k):
        pltpu.make_async_copy(x_hbm.at[pl.ds(0, CH)], bufs_a[b], ssem_a).wait()
        pltpu.make_async_copy(x_hbm.at[pl.ds(0, CH)], bufs_b[b], ssem_b).wait()


def _agg_count_body(x_hbm, src_hbm, dst_hbm, part_hbm, cnt_hbm,
                    idx_s, idx_d, zed, ones_v,
                    a0, a1, a2, a3, a4, b0, b1, b2, b3, b4,
                    acc_sh, cnt_sh, gsem, ssem_a, ssem_b, csem):
    c = lax.axis_index("c")
    s = lax.axis_index("s")
    w = c * NS + s

    _fill_1d(zed, APS, 0.0)
    _fill_1d(ones_v, 128, 1.0)
    pltpu.sync_copy(zed, cnt_sh.at[pl.ds(s * APS, APS)])
    _fill_2d_bf16(a0, CH, D, 0.0)
    _zero_acc(a0, acc_sh, s)

    pltpu.sync_copy(src_hbm.at[w], idx_s)
    pltpu.sync_copy(dst_hbm.at[w], idx_d)

    plsc.subcore_barrier()

    def cfire(j, carry):
        pltpu.async_copy(ones_v.at[pl.ds(0, CH)], cnt_sh.at[idx_d.at[j]],
                         csem, add=True)
        return carry

    lax.fori_loop(0, NCH, cfire, 0)

    _agg_pipeline(x_hbm, idx_s, idx_d, (a0, a1, a2, a3, a4),
                  (b0, b1, b2, b3, b4), acc_sh, gsem, ssem_a, ssem_b, KA)

    def cdrain(j, carry):
        pltpu.make_async_copy(cnt_hbm.at[pl.ds(0, CH)], zed.at[pl.ds(0, CH)],
                              csem).wait()
        return carry

    lax.fori_loop(0, NCH, cdrain, 0)

    plsc.subcore_barrier()

    pltpu.sync_copy(acc_sh.at[pl.ds(s * RPS, RPS)],
                    part_hbm.at[c, pl.ds(s * RPS, RPS)])
    pltpu.sync_copy(cnt_sh.at[pl.ds(s * RPS, RPS)],
                    cnt_hbm.at[pl.ds(c * N_OUT + s * RPS, RPS)])


def _agg_body(x_hbm, src_hbm, dst_hbm, part_hbm,
              idx_s, idx_d, a0, a1, a2, a3, a4, b0, b1, b2, b3, b4, acc_sh,
              gsem, ssem_a, ssem_b):
    c = lax.axis_index("c")
    s = lax.axis_index("s")
    w = c * NS + s

    _fill_2d_bf16(a0, CH, D, 0.0)
    _zero_acc(a0, acc_sh, s)

    pltpu.sync_copy(src_hbm.at[w], idx_s)
    pltpu.sync_copy(dst_hbm.at[w], idx_d)

    plsc.subcore_barrier()

    _agg_pipeline(x_hbm, idx_s, idx_d, (a0, a1, a2, a3, a4),
                  (b0, b1, b2, b3, b4), acc_sh, gsem, ssem_a, ssem_b, KA)

    plsc.subcore_barrier()

    pltpu.sync_copy(acc_sh.at[pl.ds(s * RPS, RPS)],
                    part_hbm.at[c, pl.ds(s * RPS, RPS)])


def _row_buf():
    return pltpu.VMEM((CH, D), BF)


_agg_count = functools.partial(
    pl.kernel,
    out_type=[jax.ShapeDtypeStruct((NC, N_OUT, D), BF),
              jax.ShapeDtypeStruct((NC * N_OUT,), jnp.float32)],
    mesh=_mesh(),
    compiler_params=_SC_PARAMS,
    scratch_types=[
        pltpu.VMEM((NCH, CH), jnp.int32),
        pltpu.VMEM((NCH, CH), jnp.int32),
        pltpu.VMEM((APS,), jnp.float32),
        pltpu.VMEM((128,), jnp.float32),
        _row_buf(), _row_buf(), _row_buf(), _row_buf(), _row_buf(),
        _row_buf(), _row_buf(), _row_buf(), _row_buf(), _row_buf(),
        pltpu.VMEM_SHARED((N_ACC, D), BF),
        pltpu.VMEM_SHARED((N_ACC,), jnp.float32),
        pltpu.SemaphoreType.DMA,
        pltpu.SemaphoreType.DMA,
        pltpu.SemaphoreType.DMA,
        pltpu.SemaphoreType.DMA,
    ],
)(_agg_count_body)

_agg = functools.partial(
    pl.kernel,
    out_type=jax.ShapeDtypeStruct((NC, N_OUT, D), BF),
    mesh=_mesh(),
    compiler_params=_SC_PARAMS,
    scratch_types=[
        pltpu.VMEM((NCH, CH), jnp.int32),
        pltpu.VMEM((NCH, CH), jnp.int32),
        _row_buf(), _row_buf(), _row_buf(), _row_buf(), _row_buf(),
        _row_buf(), _row_buf(), _row_buf(), _row_buf(), _row_buf(),
        pltpu.VMEM_SHARED((N_ACC, D), BF),
        pltpu.SemaphoreType.DMA,
        pltpu.SemaphoreType.DMA,
        pltpu.SemaphoreType.DMA,
    ],
)(_agg_body)


def _dot_chunk(bufS, bufD, out_v, j, tbuf):
    row_iota = lax.iota(jnp.int32, 16)

    def grp(g, carry):
        for e_ in range(16):
            e = g * 16 + e_
            acc = jnp.zeros((16,), jnp.float32)
            for k in range(D // 32):
                s32 = bufS[e, pl.ds(k * 32, 32)]
                d32 = bufD[e, pl.ds(k * 32, 32)]
                s0, s1 = plsc.unpack(s32, format=plsc.PackFormat.INTERLEAVED)
                d0, d1 = plsc.unpack(d32, format=plsc.PackFormat.INTERLEAVED)
                acc = acc + s0 * d0 + s1 * d1
            tbuf[e_, pl.ds(0, 16)] = acc
        tot = jnp.zeros((16,), jnp.float32)
        for col in range(16):
            colv = jnp.full((16,), col, jnp.int32)
            tot = tot + plsc.load_gather(tbuf, [row_iota, colv])
        out_v[pl.ds(j * SCH2 + g * 16, 16)] = tot
        return carry

    lax.fori_loop(0, SCH2 // 16, grp, 0)


def _sup_score_body(h_hbm, ssrc_hbm, sdst_hbm, out_hbm,
                    idx_s, idx_d, out_v, tbuf, aS, aD, bS, bD,
                    gsem_a, gsem_b):
    c = lax.axis_index("c")
    s = lax.axis_index("s")
    w = c * NS + s

    pltpu.sync_copy(ssrc_hbm.at[w], idx_s)
    pltpu.sync_copy(sdst_hbm.at[w], idx_d)

    def fire(j, bS_, bD_, sem):
        pltpu.async_copy(h_hbm.at[idx_s.at[j]], bS_, sem)
        pltpu.async_copy(h_hbm.at[idx_d.at[j]], bD_, sem)

    def drain(bS_, bD_, sem):
        pltpu.make_async_copy(h_hbm.at[pl.ds(0, SCH2)], bS_, sem).wait()
        pltpu.make_async_copy(h_hbm.at[pl.ds(0, SCH2)], bD_, sem).wait()

    npairs = NSCH // 2
    fire(0, aS, aD, gsem_a)

    def pair(t, carry):
        fire(2 * t + 1, bS, bD, gsem_b)
        drain(aS, aD, gsem_a)
        _dot_chunk(aS, aD, out_v, 2 * t, tbuf)

        @pl.when(t < npairs - 1)
        def _():
            fire(2 * t + 2, aS, aD, gsem_a)
        drain(bS, bD, gsem_b)
        _dot_chunk(bS, bD, out_v, 2 * t + 1, tbuf)
        return carry

    lax.fori_loop(0, npairs, pair, 0)
    pltpu.sync_copy(out_v, out_hbm.at[pl.ds(w * SPW, SPW)])


_sup_score = functools.partial(
    pl.kernel,
    out_type=jax.ShapeDtypeStruct((E_SUP_PAD,), jnp.float32),
    mesh=_mesh(),
    compiler_params=_SC_PARAMS_NL,
    scratch_types=[
        pltpu.VMEM((NSCH, SCH2), jnp.int32),
        pltpu.VMEM((NSCH, SCH2), jnp.int32),
        pltpu.VMEM((SPW,), jnp.float32),
        pltpu.VMEM((16, 16), jnp.float32),
        pltpu.VMEM((SCH2, D), BF),
        pltpu.VMEM((SCH2, D), BF),
        pltpu.VMEM((SCH2, D), BF),
        pltpu.VMEM((SCH2, D), BF),
        pltpu.SemaphoreType.DMA,
        pltpu.SemaphoreType.DMA,
    ],
)(_sup_score_body)


def _sage_tc(part, cnt2, x, Wl, bl, Wr, relu, emit_bf16):
    def body(part_any, cnt_ref, x_ref, wl_ref, bl_ref, wr_ref, *rest):
        if emit_bf16:
            h_out, hb_any, pvm, hbvm = rest
        else:
            h_out, pvm = rest
        pltpu.sync_copy(part_any, pvm)
        cnt = cnt_ref[:, 0] + cnt_ref[:, 1]
        recip = 1.0 / jnp.maximum(cnt, 1.0)
        agg = (pvm[0, :N, :].astype(jnp.float32)
               + pvm[1, :N, :].astype(jnp.float32))
        mean = agg * recip[:, None]
        h = lax.dot_general(mean, wl_ref[...], (((1,), (1,)), ((), ())),
                            preferred_element_type=jnp.float32)
        h = h + bl_ref[...]
        h = h + lax.dot_general(x_ref[...], wr_ref[...], (((1,), (1,)), ((), ())),
                                preferred_element_type=jnp.float32)
        if relu:
            h = jnp.maximum(h, 0.0)
        h_out[...] = h
        if emit_bf16:
            hbvm[...] = h.astype(BF)
            pltpu.sync_copy(hbvm, hb_any)

    out_specs = [pl.BlockSpec((N, D), lambda: (0, 0))]
    out_shape = [jax.ShapeDtypeStruct((N, D), jnp.float32)]
    scratch = [pltpu.VMEM((NC, N_OUT, D), BF)]
    if emit_bf16:
        out_specs.append(pl.BlockSpec(memory_space=pl.ANY))
        out_shape.append(jax.ShapeDtypeStruct((N, D), BF))
        scratch.append(pltpu.VMEM((N, D), BF))
    return pl.pallas_call(
        body,
        in_specs=[
            pl.BlockSpec(memory_space=pl.ANY),
            pl.BlockSpec((N, NC), lambda: (0, 0)),
            pl.BlockSpec((N, D), lambda: (0, 0)),
            pl.BlockSpec((D, D), lambda: (0, 0)),
            pl.BlockSpec((1, D), lambda: (0, 0)),
            pl.BlockSpec((D, D), lambda: (0, 0)),
        ],
        out_specs=out_specs,
        out_shape=out_shape,
        scratch_shapes=scratch,
    )(part, cnt2, x, Wl, bl, Wr)


def kernel(node_embeddings, message_passing_edge_index, supervision_edge_index,
           Wl1, bl1, Wr1, Wl2, bl2, Wr2, Wl3, bl3, Wr3):
    src = message_passing_edge_index[0].reshape(NW, NCH, CH)
    dst = message_passing_edge_index[1].reshape(NW, NCH, CH)
    pad_idx = jnp.arange(E_SUP_PAD - E_SUP, dtype=jnp.int32) % N
    sup_pad = jnp.concatenate(
        [supervision_edge_index, jnp.stack([pad_idx, pad_idx])], axis=1)
    ssrc = sup_pad[0].reshape(NW, NSCH, SCH2)
    sdst = sup_pad[1].reshape(NW, NSCH, SCH2)

    x = node_embeddings
    xb = x.astype(BF)

    part, cnt = _agg_count(xb, src, dst)
    cnt2 = cnt.reshape(NC, N_OUT)[:, :N].T

    h, hb = _sage_tc(part, cnt2, x, Wl1, bl1.reshape(1, D), Wr1, True, True)
    part = _agg(hb, src, dst)
    h, hb = _sage_tc(part, cnt2, h, Wl2, bl2.reshape(1, D), Wr2, True, True)
    part = _agg(hb, src, dst)
    hb = _sage_tc(part, cnt2, h, Wl3, bl3.reshape(1, D), Wr3, False, True)[1]

    scores = _sup_score(hb, ssrc, sdst)
    return scores[:E_SUP]
